# baseline (speedup 1.0000x reference)
import functools

import jax
import jax.numpy as jnp
from jax import lax
from jax.experimental import pallas as pl
from jax.experimental.pallas import tpu as pltpu
from jax.experimental.pallas import tpu_sc as plsc

B, L, D, E = 4096, 77, 64, 8
N = B * L
NW = 32
IDS_PER_W = N // NW
CHUNK = 112
NCHUNK = IDS_PER_W // CHUNK
NBUF = 11
LOOK = 6
LAG = NBUF - LOOK
SCAT = 128
NSCAT = (B * E) // (NW * SCAT)


def kernel(input_ids, fix_vec, fix_offsets, table):
    ids_r = input_ids.reshape(NW, NCHUNK, CHUNK)
    pos = (jnp.arange(B, dtype=jnp.int32) * L + fix_offsets + 1)[:, None] \
        + jnp.arange(E, dtype=jnp.int32)[None, :]
    pos_r = pos.reshape(NW, NSCAT, SCAT)
    fix_tiled = jnp.tile(fix_vec, (SCAT // E, 1))

    mesh = plsc.VectorSubcoreMesh(core_axis_name="c", subcore_axis_name="s")

    @functools.partial(
        pl.kernel, mesh=mesh,
        compiler_params=pltpu.CompilerParams(use_tc_tiling_on_sc=False),
        out_type=jax.ShapeDtypeStruct((N, D), jnp.float32),
        scratch_types=[
            pltpu.VMEM((NCHUNK, CHUNK), jnp.int32),
            pltpu.VMEM((NBUF, CHUNK, D), jnp.float32),
            pltpu.VMEM((NSCAT, SCAT), jnp.int32),
            pltpu.VMEM((SCAT, D), jnp.float32),
            pltpu.SemaphoreType.DMA((NBUF,)),
            pltpu.SemaphoreType.DMA((NBUF,)),
            pltpu.SemaphoreType.DMA,
        ],
    )
    def emb_fix_kernel(ids_hbm, pos_hbm, fixt_hbm, table_hbm, out_hbm,
                       idx_v, rows_v, pos_v, fixt_v, gsem, wsem, ssem):
        wid = lax.axis_index("s") * 2 + lax.axis_index("c")
        pltpu.sync_copy(ids_hbm.at[wid], idx_v)
        base = wid * IDS_PER_W

        def g_copy(j, b):
            return pltpu.make_async_copy(
                table_hbm.at[idx_v.at[j]], rows_v.at[b], gsem.at[b])

        def w_copy(j, b):
            return pltpu.make_async_copy(
                rows_v.at[b], out_hbm.at[pl.ds(base + j * CHUNK, CHUNK)],
                wsem.at[b])

        for j in range(LOOK):
            g_copy(j, j).start()
        for j in range(NBUF):
            g_copy(j, j % NBUF).wait()
            if j >= LAG:
                w_copy(j - LAG, (j + LOOK) % NBUF).wait()
            g_copy(j + LOOK, (j + LOOK) % NBUF).start()
            w_copy(j, j % NBUF).start()

        @pl.loop(1, NCHUNK // NBUF - 1)
        def _(s):
            j0 = s * NBUF
            for k in range(NBUF):
                j = j0 + k
                g_copy(j, k).wait()
                w_copy(j - LAG, (k + LOOK) % NBUF).wait()
                g_copy(j + LOOK, (k + LOOK) % NBUF).start()
                w_copy(j, k).start()

        j0 = NCHUNK - NBUF
        for k in range(NBUF):
            j = j0 + k
            g_copy(j, k).wait()
            if k < LAG:
                w_copy(j - LAG, (k + LOOK) % NBUF).wait()
                g_copy(j + LOOK, (k + LOOK) % NBUF).start()
            w_copy(j, k).start()
        for k in range(NBUF):
            w_copy(j0 + k, k).wait()

        pltpu.sync_copy(pos_hbm.at[wid], pos_v)
        pltpu.sync_copy(fixt_hbm, fixt_v)
        for j in range(NSCAT):
            pltpu.async_copy(fixt_v, out_hbm.at[pos_v.at[j]], ssem)
        for j in range(NSCAT):
            pltpu.make_async_copy(fixt_v, out_hbm.at[pos_v.at[j]], ssem).wait()

    out = emb_fix_kernel(ids_r, pos_r, fix_tiled, table)
    return out.reshape(B, L, D)
